# baseline (speedup 1.0000x reference)
import functools

import jax
import jax.numpy as jnp
from jax.experimental import pallas as pl
from jax.experimental.pallas import tpu as pltpu

THRESH = 0.5
ALPHA = 0.1

_BK = 512
_W = 1360
_RPG = 16


def _body(x_ref, y_ref, s_ref, o_ref, *, g_total):
    i = pl.program_id(0)

    @pl.when(i == 0)
    def _():
        o_ref[...] = jnp.zeros_like(o_ref)

    xb = x_ref[...]
    yb = y_ref[...]
    s_sum = s_ref[:, 0:16]
    s_ex = s_ref[:, 16:32]

    t = yb - xb
    p = t * t
    q = xb * xb

    ps_p = jnp.dot(p, s_sum, preferred_element_type=jnp.float32)
    ps_q = jnp.dot(q, s_sum, preferred_element_type=jnp.float32)
    x0sq = jnp.dot(q, s_ex, preferred_element_type=jnp.float32)
    y0 = jnp.dot(yb, s_ex, preferred_element_type=jnp.float32)

    g = i * _BK + jax.lax.broadcasted_iota(jnp.int32, (_BK, _RPG), 0)
    valid = g < g_total
    m = jnp.logical_and(y0 > THRESH, valid)
    o_ref[0] += jnp.where(m, ps_p, 0.0)
    o_ref[1] += jnp.where(m, ps_q, 0.0)
    o_ref[2] += jnp.where(valid, x0sq, 0.0)


def kernel(x, y):
    B, N, C = x.shape
    total = B * N * C
    g_total = total // _W
    xf = x.reshape(g_total, _W)
    yf = y.reshape(g_total, _W)

    e = jnp.arange(_W, dtype=jnp.int32)
    j = jnp.arange(_RPG, dtype=jnp.int32)
    s_sum = (e[:, None] // C == j[None, :]).astype(jnp.float32)
    s_ex = (e[:, None] == j[None, :] * C).astype(jnp.float32)
    s_all = jnp.concatenate([s_sum, s_ex], axis=1)

    ng = (g_total + _BK - 1) // _BK
    out = pl.pallas_call(
        functools.partial(_body, g_total=g_total),
        grid=(ng,),
        in_specs=[
            pl.BlockSpec((_BK, _W), lambda i: (i, 0)),
            pl.BlockSpec((_BK, _W), lambda i: (i, 0)),
            pl.BlockSpec((_W, 32), lambda i: (0, 0)),
        ],
        out_specs=pl.BlockSpec((3, _BK, _RPG), lambda i: (0, 0, 0)),
        out_shape=jax.ShapeDtypeStruct((3, _BK, _RPG), jnp.float32),
    )(xf, yf, s_all)
    return jnp.sum(out[0]) - ALPHA * (jnp.sum(out[1]) - jnp.sum(out[2]))


# speedup vs baseline: 11.1443x; 11.1443x over previous
import jax
import jax.numpy as jnp
from jax.experimental import pallas as pl

THRESH = 0.5
ALPHA = 0.1


def _body(x_ref, y_ref, o_ref):
    @pl.when(pl.program_id(0) == 0)
    def _():
        o_ref[...] = jnp.zeros_like(o_ref)

    xv = x_ref[0]
    yv = y_ref[0]
    mf = (yv[:, 0:1] > THRESH).astype(jnp.float32)
    t = yv - xv
    u = t * mf
    v = xv * mf
    o_ref[0, :] += jnp.sum(u * t, axis=0)
    o_ref[1, :] += jnp.sum(v * xv, axis=0)
    o_ref[2, :] += jnp.sum(xv * xv, axis=0)


def kernel(x, y):
    B, N, C = x.shape
    out = pl.pallas_call(
        _body,
        grid=(B,),
        in_specs=[
            pl.BlockSpec((1, N, C), lambda i: (i, 0, 0)),
            pl.BlockSpec((1, N, C), lambda i: (i, 0, 0)),
        ],
        out_specs=pl.BlockSpec((3, C), lambda i: (0, 0)),
        out_shape=jax.ShapeDtypeStruct((3, C), jnp.float32),
    )(x, y)
    return (jnp.sum(out[0]) - ALPHA * jnp.sum(out[1])
            + ALPHA * out[2, 0])


# speedup vs baseline: 11.3220x; 1.0159x over previous
import jax
import jax.numpy as jnp
from jax import lax
from jax.experimental import pallas as pl
from jax.experimental.pallas import tpu as pltpu

THRESH = 0.5
ALPHA = 0.1
_NS = 4


def _start(x_hbm, y_hbm, xb, yb, sx, sy, b):
    s = lax.rem(b, _NS)
    pltpu.make_async_copy(x_hbm.at[b], xb.at[s], sx.at[s]).start()
    pltpu.make_async_copy(y_hbm.at[b], yb.at[s], sy.at[s]).start()


def _body(x_hbm, y_hbm, o_ref, xb, yb, sx, sy):
    B = x_hbm.shape[0]
    o_ref[...] = jnp.zeros_like(o_ref)
    for b in range(_NS):
        _start(x_hbm, y_hbm, xb, yb, sx, sy, jnp.int32(b))

    def step(b, carry):
        s = lax.rem(b, _NS)
        pltpu.make_async_copy(x_hbm.at[b], xb.at[s], sx.at[s]).wait()
        pltpu.make_async_copy(y_hbm.at[b], yb.at[s], sy.at[s]).wait()
        xv = xb[s]
        yv = yb[s]
        mf = (yv[:, 0:1] > THRESH).astype(jnp.float32)
        t = yv - xv
        u = t * mf
        v = xv * mf
        o_ref[0, :] += jnp.sum(u * t, axis=0)
        o_ref[1, :] += jnp.sum(v * xv, axis=0)
        o_ref[2, :] += jnp.sum(xv * xv, axis=0)

        @pl.when(b + _NS < B)
        def _():
            _start(x_hbm, y_hbm, xb, yb, sx, sy, b + _NS)

        return carry

    lax.fori_loop(0, B, step, 0)


def kernel(x, y):
    B, N, C = x.shape
    out = pl.pallas_call(
        _body,
        in_specs=[
            pl.BlockSpec(memory_space=pltpu.HBM),
            pl.BlockSpec(memory_space=pltpu.HBM),
        ],
        out_specs=pl.BlockSpec(memory_space=pltpu.VMEM),
        out_shape=jax.ShapeDtypeStruct((3, C), jnp.float32),
        scratch_shapes=[
            pltpu.VMEM((_NS, N, C), jnp.float32),
            pltpu.VMEM((_NS, N, C), jnp.float32),
            pltpu.SemaphoreType.DMA((_NS,)),
            pltpu.SemaphoreType.DMA((_NS,)),
        ],
    )(x, y)
    return (jnp.sum(out[0]) - ALPHA * jnp.sum(out[1])
            + ALPHA * out[2, 0])


# speedup vs baseline: 11.3481x; 1.0023x over previous
import jax
import jax.numpy as jnp
from jax import lax
from jax.experimental import pallas as pl
from jax.experimental.pallas import tpu as pltpu

THRESH = 0.5
ALPHA = 0.1
_NS = 5


def _start(x_hbm, y_hbm, xb, yb, sx, sy, b):
    s = lax.rem(b, _NS)
    pltpu.make_async_copy(x_hbm.at[b], xb.at[s], sx.at[s]).start()
    pltpu.make_async_copy(y_hbm.at[b], yb.at[s], sy.at[s]).start()


def _body(x_hbm, y_hbm, o_ref, xb, yb, sx, sy):
    B = x_hbm.shape[0]
    o_ref[...] = jnp.zeros_like(o_ref)
    for b in range(_NS):
        _start(x_hbm, y_hbm, xb, yb, sx, sy, jnp.int32(b))

    def step(b, carry):
        s = lax.rem(b, _NS)
        pltpu.make_async_copy(x_hbm.at[b], xb.at[s], sx.at[s]).wait()
        pltpu.make_async_copy(y_hbm.at[b], yb.at[s], sy.at[s]).wait()
        xv = xb[s]
        yv = yb[s]
        mf = (yv[:, 0:1] > THRESH).astype(jnp.float32)
        t = yv - xv
        u = t * mf
        v = xv * mf
        o_ref[0, :] += jnp.sum(u * t, axis=0)
        o_ref[1, :] += jnp.sum(v * xv, axis=0)
        o_ref[2, :] += jnp.sum(xv * xv, axis=0)

        @pl.when(b + _NS < B)
        def _():
            _start(x_hbm, y_hbm, xb, yb, sx, sy, b + _NS)

        return carry

    lax.fori_loop(0, B, step, 0)


def kernel(x, y):
    B, N, C = x.shape
    out = pl.pallas_call(
        _body,
        in_specs=[
            pl.BlockSpec(memory_space=pltpu.HBM),
            pl.BlockSpec(memory_space=pltpu.HBM),
        ],
        out_specs=pl.BlockSpec(memory_space=pltpu.VMEM),
        out_shape=jax.ShapeDtypeStruct((3, C), jnp.float32),
        scratch_shapes=[
            pltpu.VMEM((_NS, N, C), jnp.float32),
            pltpu.VMEM((_NS, N, C), jnp.float32),
            pltpu.SemaphoreType.DMA((_NS,)),
            pltpu.SemaphoreType.DMA((_NS,)),
        ],
    )(x, y)
    return (jnp.sum(out[0]) - ALPHA * jnp.sum(out[1])
            + ALPHA * out[2, 0])
